# baseline (speedup 1.0000x reference)
import functools

import jax
import jax.numpy as jnp
from jax import lax
from jax.experimental import pallas as pl
from jax.experimental.pallas import tpu as pltpu
from jax.experimental.pallas import tpu_sc as plsc

N = 10000
E = 320000
D = 128
H = 128
OUT = 512
G = 64

N_PAD = 10240
CHUNK = 128
NW = 32
E_PAD = 327680
EPW = E_PAD // NW
NCHUNKS = EPW // CHUNK
NBUF = 2
ROWS_PER_SUB = N_PAD // 16
CA = 124
CB = 36
TOTAL_CHUNKS = E_PAD // CHUNK


MAXHALF = max(CA, CB) // 2


def _sc_agg(h_pad, e2, zeros_pad):
    mesh = plsc.VectorSubcoreMesh(core_axis_name="c", subcore_axis_name="s")

    @functools.partial(
        pl.kernel,
        mesh=mesh,
        out_type=jax.ShapeDtypeStruct((2, N_PAD, D), jnp.float32),
        scratch_types=[
            pltpu.VMEM((MAXHALF, 2, CHUNK), jnp.int32),
            pltpu.VMEM((NBUF, CHUNK, D), jnp.float32),
            pltpu.VMEM_SHARED((N_PAD, D), jnp.float32),
            pltpu.SemaphoreType.DMA((NBUF,)),
        ],
    )
    def k(h_hbm, e2_hbm, z_hbm, out_hbm, ibuf, rows, acc, gsem):
        c = lax.axis_index("c")
        s = lax.axis_index("s")
        pltpu.sync_copy(
            z_hbm.at[c, pl.ds(s * ROWS_PER_SUB, ROWS_PER_SUB)],
            acc.at[pl.ds(s * ROWS_PER_SUB, ROWS_PER_SUB)],
        )
        plsc.subcore_barrier()

        def gather(i, b):
            pltpu.async_copy(h_hbm.at[ibuf.at[i, 0]], rows.at[b], gsem.at[b])

        def gwait(i, b):
            pltpu.make_async_copy(h_hbm.at[ibuf.at[i, 0]], rows.at[b],
                                  gsem.at[b]).wait()

        def pipeline(base, nch):
            half = nch // 2
            for q in range(2):
                pltpu.sync_copy(e2_hbm.at[pl.ds(base + q * half, half)],
                                ibuf.at[pl.ds(0, half)])
                for b in range(NBUF):
                    gather(b, b)

                def body(j, carry):
                    i0 = j * NBUF
                    for b in range(NBUF):
                        i = i0 + b
                        gwait(i, b)
                        pltpu.sync_copy(rows.at[b], acc.at[ibuf.at[i, 1]],
                                        add=True)
                        nxt = i + NBUF
                        nxt = jnp.where(nxt >= half, nxt - half, nxt)
                        gather(nxt, b)
                    return carry

                lax.fori_loop(0, half // NBUF, body, 0)
                for b in range(NBUF):
                    gwait(b, b)

        @pl.when(c == 0)
        def _core0():
            pipeline(s * CA, CA)

        @pl.when(c == 1)
        def _core1():
            pipeline(16 * CA + s * CB, CB)

        plsc.subcore_barrier()
        pltpu.sync_copy(
            acc.at[pl.ds(s * ROWS_PER_SUB, ROWS_PER_SUB)],
            out_hbm.at[c, pl.ds(s * ROWS_PER_SUB, ROWS_PER_SUB)],
        )

    return k(h_pad, e2, zeros_pad)


def _ln(x, g, b):
    m = jnp.mean(x, axis=-1, keepdims=True)
    v = jnp.mean((x - m) * (x - m), axis=-1, keepdims=True)
    return (x - m) * lax.rsqrt(v + 1e-5) * g + b


def _mlp_body(h_ref, a_ref, eps_ref, w1_ref, g1_ref, b1_ref, w2_ref, g2_ref,
              b2_ref, ng_ref, nb_ref, out_ref):
    eps = eps_ref[0, 0]
    u = (1.0 + eps) * h_ref[...] + a_ref[0] + a_ref[1]
    z1 = jnp.dot(u, w1_ref[...], preferred_element_type=jnp.float32)
    t = jnp.maximum(_ln(z1, g1_ref[...], b1_ref[...]), 0.0)
    z2 = jnp.dot(t, w2_ref[...], preferred_element_type=jnp.float32)
    z2 = _ln(z2, g2_ref[...], b2_ref[...])
    out_ref[...] = jnp.maximum(_ln(z2, ng_ref[...], nb_ref[...]), 0.0)


def _tc_mlp(h_pad, aggs, eps, w1, g1, b1, w2, g2, b2, ng, nb):
    blk = 1024
    grid = N_PAD // blk
    row = lambda v: v.reshape(1, H)
    return pl.pallas_call(
        _mlp_body,
        grid=(grid,),
        in_specs=[
            pl.BlockSpec((blk, D), lambda i: (i, 0)),
            pl.BlockSpec((2, blk, D), lambda i: (0, i, 0)),
            pl.BlockSpec(memory_space=pltpu.SMEM),
            pl.BlockSpec((D, H), lambda i: (0, 0)),
            pl.BlockSpec((1, H), lambda i: (0, 0)),
            pl.BlockSpec((1, H), lambda i: (0, 0)),
            pl.BlockSpec((H, H), lambda i: (0, 0)),
            pl.BlockSpec((1, H), lambda i: (0, 0)),
            pl.BlockSpec((1, H), lambda i: (0, 0)),
            pl.BlockSpec((1, H), lambda i: (0, 0)),
            pl.BlockSpec((1, H), lambda i: (0, 0)),
        ],
        out_specs=pl.BlockSpec((blk, H), lambda i: (i, 0)),
        out_shape=jax.ShapeDtypeStruct((N_PAD, H), jnp.float32),
    )(h_pad, aggs, eps.reshape(1, 1), w1, row(g1), row(b1), w2, row(g2),
      row(b2), row(ng), row(nb))


def _pool_body(h_ref, b_ref, wp_ref, lpg_ref, lpb_ref, out_ref, s_scr, c_scr):
    i = pl.program_id(0)
    nsteps = pl.num_programs(0)

    @pl.when(i == 0)
    def _init():
        s_scr[...] = jnp.zeros_like(s_scr)
        c_scr[...] = jnp.zeros_like(c_scr)

    b = b_ref[0, 0, :]
    blk = b.shape[0]
    onehot = (b[:, None] == lax.broadcasted_iota(jnp.int32, (blk, G), 1))
    onehot = onehot.astype(jnp.float32)
    dn = (((0,), (0,)), ((), ()))
    s_scr[...] += lax.dot_general(onehot, h_ref[...], dn,
                                  preferred_element_type=jnp.float32)
    c_scr[...] += lax.dot_general(onehot, jnp.ones_like(h_ref[...]), dn,
                                  preferred_element_type=jnp.float32)

    @pl.when(i == nsteps - 1)
    def _final():
        hp = s_scr[...] / jnp.maximum(c_scr[...], 1.0)
        y = jnp.dot(hp, wp_ref[...], preferred_element_type=jnp.float32)
        out_ref[...] = _ln(y, lpg_ref[...], lpb_ref[...])


def _tc_pool(h_pad, batch_pad, wp, lpg, lpb):
    blk = 1024
    grid = N_PAD // blk
    b3 = batch_pad.reshape(grid, 1, blk)
    return pl.pallas_call(
        _pool_body,
        grid=(grid,),
        in_specs=[
            pl.BlockSpec((blk, H), lambda i: (i, 0)),
            pl.BlockSpec((1, 1, blk), lambda i: (i, 0, 0)),
            pl.BlockSpec((H, OUT), lambda i: (0, 0)),
            pl.BlockSpec((1, OUT), lambda i: (0, 0)),
            pl.BlockSpec((1, OUT), lambda i: (0, 0)),
        ],
        out_specs=pl.BlockSpec((G, OUT), lambda i: (0, 0)),
        out_shape=jax.ShapeDtypeStruct((G, OUT), jnp.float32),
        scratch_shapes=[
            pltpu.VMEM((G, H), jnp.float32),
            pltpu.VMEM((G, H), jnp.float32),
        ],
    )(h_pad, b3, wp, lpg.reshape(1, OUT), lpb.reshape(1, OUT))


def kernel(x, edge_index, batch,
           eps0, W1_0, ln1g_0, ln1b_0, W2_0, ln2g_0, ln2b_0, ng_0, nb_0,
           eps1, W1_1, ln1g_1, ln1b_1, W2_1, ln2g_1, ln2b_1, ng_1, nb_1,
           eps2, W1_2, ln1g_2, ln1b_2, W2_2, ln2g_2, ln2b_2, ng_2, nb_2,
           Wp, lpg, lpb):
    src = edge_index[0]
    dst = edge_index[1]
    pad_e = E_PAD - E
    src_pad = jnp.concatenate([src, jnp.zeros((pad_e,), jnp.int32)])
    dst_pad = jnp.concatenate([dst, jnp.full((pad_e,), N, jnp.int32)])
    e2 = jnp.stack([src_pad.reshape(TOTAL_CHUNKS, CHUNK),
                    dst_pad.reshape(TOTAL_CHUNKS, CHUNK)], axis=1)
    h = jnp.pad(x, ((0, N_PAD - N), (0, 0)))
    zeros_pad = jnp.zeros((2, N_PAD, D), jnp.float32)
    batch_pad = jnp.concatenate([batch, jnp.full((N_PAD - N,), G, jnp.int32)])

    params = [
        (eps0, W1_0, ln1g_0, ln1b_0, W2_0, ln2g_0, ln2b_0, ng_0, nb_0),
        (eps1, W1_1, ln1g_1, ln1b_1, W2_1, ln2g_1, ln2b_1, ng_1, nb_1),
        (eps2, W1_2, ln1g_2, ln1b_2, W2_2, ln2g_2, ln2b_2, ng_2, nb_2),
    ]
    for (eps, w1, g1, b1, w2, g2, b2, ng, nb) in params:
        aggs = _sc_agg(h, e2, zeros_pad)
        h = _tc_mlp(h, aggs, eps, w1, g1, b1, w2, g2, b2, ng, nb)
    return _tc_pool(h, batch_pad, Wp, lpg, lpb)


# speedup vs baseline: 1.0757x; 1.0757x over previous
import functools

import jax
import jax.numpy as jnp
from jax import lax
from jax.experimental import pallas as pl
from jax.experimental.pallas import tpu as pltpu
from jax.experimental.pallas import tpu_sc as plsc

N = 10000
E = 320000
D = 128
H = 128
OUT = 512
G = 64

N_PAD = 10240
CHUNK = 128
NW = 32
E_PAD = 327680
EPW = E_PAD // NW
NCHUNKS = EPW // CHUNK
NBUF = 2
ROWS_PER_SUB = N_PAD // 16
CA = 124
CB = 36
TOTAL_CHUNKS = E_PAD // CHUNK


MAXHALF = max(CA, CB) // 2


def _sc_agg(h_pad, e2, zeros_pad):
    mesh = plsc.VectorSubcoreMesh(core_axis_name="c", subcore_axis_name="s")

    @functools.partial(
        pl.kernel,
        mesh=mesh,
        out_type=jax.ShapeDtypeStruct((2, N_PAD, D), jnp.float32),
        scratch_types=[
            pltpu.VMEM((MAXHALF, 2, CHUNK), jnp.int32),
            pltpu.VMEM((NBUF, CHUNK, D), jnp.float32),
            pltpu.VMEM_SHARED((N_PAD, D), jnp.float32),
            pltpu.SemaphoreType.DMA((NBUF,)),
        ],
    )
    def k(h_hbm, e2_hbm, z_hbm, out_hbm, ibuf, rows, acc, gsem):
        c = lax.axis_index("c")
        s = lax.axis_index("s")
        with jax.named_scope("agg_zero"):
            pltpu.sync_copy(
                z_hbm.at[pl.ds(s * ROWS_PER_SUB, ROWS_PER_SUB)],
                acc.at[pl.ds(s * ROWS_PER_SUB, ROWS_PER_SUB)],
            )
            plsc.subcore_barrier()

        def gather(i, b):
            pltpu.async_copy(h_hbm.at[ibuf.at[i, 0]], rows.at[b], gsem.at[b])

        def gwait(i, b):
            pltpu.make_async_copy(h_hbm.at[ibuf.at[i, 0]], rows.at[b],
                                  gsem.at[b]).wait()

        def pipeline(base, nch):
            half = nch // 2
            for q in range(2):
                pltpu.sync_copy(e2_hbm.at[pl.ds(base + q * half, half)],
                                ibuf.at[pl.ds(0, half)])
                for b in range(NBUF):
                    gather(b, b)

                def body(j, carry):
                    i0 = j * NBUF
                    for b in range(NBUF):
                        i = i0 + b
                        gwait(i, b)
                        pltpu.sync_copy(rows.at[b], acc.at[ibuf.at[i, 1]],
                                        add=True)
                        nxt = i + NBUF
                        nxt = jnp.where(nxt >= half, nxt - half, nxt)
                        gather(nxt, b)
                    return carry

                lax.fori_loop(0, half // NBUF, body, 0)
                for b in range(NBUF):
                    gwait(b, b)

        with jax.named_scope("agg_edges"):
            @pl.when(c == 0)
            def _core0():
                pipeline(s * CA, CA)

            @pl.when(c == 1)
            def _core1():
                pipeline(16 * CA + s * CB, CB)

            plsc.subcore_barrier()
        with jax.named_scope("agg_flush"):
            pltpu.sync_copy(
                acc.at[pl.ds(s * ROWS_PER_SUB, ROWS_PER_SUB)],
                out_hbm.at[c, pl.ds(s * ROWS_PER_SUB, ROWS_PER_SUB)],
            )

    return k(h_pad, e2, zeros_pad)


def _ln(x, g, b):
    m = jnp.mean(x, axis=-1, keepdims=True)
    v = jnp.mean((x - m) * (x - m), axis=-1, keepdims=True)
    return (x - m) * lax.rsqrt(v + 1e-5) * g + b


def _mlp_body(h_ref, a_ref, eps_ref, w1_ref, g1_ref, b1_ref, w2_ref, g2_ref,
              b2_ref, ng_ref, nb_ref, out_ref):
    eps = eps_ref[0, 0]
    u = (1.0 + eps) * h_ref[...] + a_ref[0] + a_ref[1]
    z1 = jnp.dot(u, w1_ref[...], preferred_element_type=jnp.float32)
    t = jnp.maximum(_ln(z1, g1_ref[...], b1_ref[...]), 0.0)
    z2 = jnp.dot(t, w2_ref[...], preferred_element_type=jnp.float32)
    z2 = _ln(z2, g2_ref[...], b2_ref[...])
    out_ref[...] = jnp.maximum(_ln(z2, ng_ref[...], nb_ref[...]), 0.0)


def _tc_mlp(h_pad, aggs, eps, w1, g1, b1, w2, g2, b2, ng, nb):
    blk = 1024
    grid = N_PAD // blk
    row = lambda v: v.reshape(1, H)
    return pl.pallas_call(
        _mlp_body,
        grid=(grid,),
        in_specs=[
            pl.BlockSpec((blk, D), lambda i: (i, 0)),
            pl.BlockSpec((2, blk, D), lambda i: (0, i, 0)),
            pl.BlockSpec(memory_space=pltpu.SMEM),
            pl.BlockSpec((D, H), lambda i: (0, 0)),
            pl.BlockSpec((1, H), lambda i: (0, 0)),
            pl.BlockSpec((1, H), lambda i: (0, 0)),
            pl.BlockSpec((H, H), lambda i: (0, 0)),
            pl.BlockSpec((1, H), lambda i: (0, 0)),
            pl.BlockSpec((1, H), lambda i: (0, 0)),
            pl.BlockSpec((1, H), lambda i: (0, 0)),
            pl.BlockSpec((1, H), lambda i: (0, 0)),
        ],
        out_specs=pl.BlockSpec((blk, H), lambda i: (i, 0)),
        out_shape=jax.ShapeDtypeStruct((N_PAD, H), jnp.float32),
    )(h_pad, aggs, eps.reshape(1, 1), w1, row(g1), row(b1), w2, row(g2),
      row(b2), row(ng), row(nb))


def _pool_body(h_ref, b_ref, wp_ref, lpg_ref, lpb_ref, out_ref, s_scr, c_scr):
    i = pl.program_id(0)
    nsteps = pl.num_programs(0)

    @pl.when(i == 0)
    def _init():
        s_scr[...] = jnp.zeros_like(s_scr)
        c_scr[...] = jnp.zeros_like(c_scr)

    b = b_ref[0, 0, :]
    blk = b.shape[0]
    onehot = (b[:, None] == lax.broadcasted_iota(jnp.int32, (blk, G), 1))
    onehot = onehot.astype(jnp.float32)
    dn = (((0,), (0,)), ((), ()))
    s_scr[...] += lax.dot_general(onehot, h_ref[...], dn,
                                  preferred_element_type=jnp.float32)
    c_scr[...] += lax.dot_general(onehot, jnp.ones_like(h_ref[...]), dn,
                                  preferred_element_type=jnp.float32)

    @pl.when(i == nsteps - 1)
    def _final():
        hp = s_scr[...] / jnp.maximum(c_scr[...], 1.0)
        y = jnp.dot(hp, wp_ref[...], preferred_element_type=jnp.float32)
        out_ref[...] = _ln(y, lpg_ref[...], lpb_ref[...])


def _tc_pool(h_pad, batch_pad, wp, lpg, lpb):
    blk = 1024
    grid = N_PAD // blk
    b3 = batch_pad.reshape(grid, 1, blk)
    return pl.pallas_call(
        _pool_body,
        grid=(grid,),
        in_specs=[
            pl.BlockSpec((blk, H), lambda i: (i, 0)),
            pl.BlockSpec((1, 1, blk), lambda i: (i, 0, 0)),
            pl.BlockSpec((H, OUT), lambda i: (0, 0)),
            pl.BlockSpec((1, OUT), lambda i: (0, 0)),
            pl.BlockSpec((1, OUT), lambda i: (0, 0)),
        ],
        out_specs=pl.BlockSpec((G, OUT), lambda i: (0, 0)),
        out_shape=jax.ShapeDtypeStruct((G, OUT), jnp.float32),
        scratch_shapes=[
            pltpu.VMEM((G, H), jnp.float32),
            pltpu.VMEM((G, H), jnp.float32),
        ],
    )(h_pad, b3, wp, lpg.reshape(1, OUT), lpb.reshape(1, OUT))


def kernel(x, edge_index, batch,
           eps0, W1_0, ln1g_0, ln1b_0, W2_0, ln2g_0, ln2b_0, ng_0, nb_0,
           eps1, W1_1, ln1g_1, ln1b_1, W2_1, ln2g_1, ln2b_1, ng_1, nb_1,
           eps2, W1_2, ln1g_2, ln1b_2, W2_2, ln2g_2, ln2b_2, ng_2, nb_2,
           Wp, lpg, lpb):
    src = edge_index[0]
    dst = edge_index[1]
    pad_e = E_PAD - E
    src_pad = jnp.concatenate([src, jnp.zeros((pad_e,), jnp.int32)])
    dst_pad = jnp.concatenate([dst, jnp.full((pad_e,), N, jnp.int32)])
    e2 = jnp.stack([src_pad.reshape(TOTAL_CHUNKS, CHUNK),
                    dst_pad.reshape(TOTAL_CHUNKS, CHUNK)], axis=1)
    h = jnp.pad(x, ((0, N_PAD - N), (0, 0)))
    zeros_pad = jnp.zeros((N_PAD, D), jnp.float32)
    batch_pad = jnp.concatenate([batch, jnp.full((N_PAD - N,), G, jnp.int32)])

    params = [
        (eps0, W1_0, ln1g_0, ln1b_0, W2_0, ln2g_0, ln2b_0, ng_0, nb_0),
        (eps1, W1_1, ln1g_1, ln1b_1, W2_1, ln2g_1, ln2b_1, ng_1, nb_1),
        (eps2, W1_2, ln1g_2, ln1b_2, W2_2, ln2g_2, ln2b_2, ng_2, nb_2),
    ]
    for (eps, w1, g1, b1, w2, g2, b2, ng, nb) in params:
        aggs = _sc_agg(h, e2, zeros_pad)
        h = _tc_mlp(h, aggs, eps, w1, g1, b1, w2, g2, b2, ng, nb)
    return _tc_pool(h, batch_pad, Wp, lpg, lpb)
